# baseline (speedup 1.0000x reference)
import functools

import jax
import jax.numpy as jnp
from jax import lax
from jax.experimental import pallas as pl
from jax.experimental.pallas import tpu as pltpu
from jax.experimental.pallas import tpu_sc as plsc

_NUM_CORES = 2
_NUM_SUBCORES = 16
_NUM_WORKERS = _NUM_CORES * _NUM_SUBCORES
_L = 16
_LAG = 4


@functools.lru_cache(maxsize=None)
def _build(B, V, D):
    b_per_w = B // _NUM_WORKERS
    n_words = b_per_w * D
    n_rows = n_words // 128
    mesh = plsc.VectorSubcoreMesh(core_axis_name="c", subcore_axis_name="s")

    @functools.partial(
        pl.kernel,
        mesh=mesh,
        out_type=jax.ShapeDtypeStruct((B * D,), jnp.float32),
        scratch_types=[
            pltpu.VMEM((b_per_w,), jnp.int32),
            pltpu.VMEM((n_rows, 128), jnp.int32),
            pltpu.VMEM((n_words,), jnp.float32),
            pltpu.SemaphoreType.DMA,
            pltpu.SemaphoreType.DMA,
            pltpu.SemaphoreType.DMA,
        ],
        compiler_params=pltpu.CompilerParams(use_tc_tiling_on_sc=False),
    )
    def gather_kernel(idx_hbm, table_hbm, out_hbm, idx_v, widx_v, land_v,
                      isem, gsem, osem):
        wid = lax.axis_index("s") * _NUM_CORES + lax.axis_index("c")
        base = wid * b_per_w
        pltpu.async_copy(idx_hbm.at[pl.ds(base, b_per_w)], idx_v, isem).wait()

        dcol = lax.iota(jnp.int32, _L) * V

        def build(g, _):
            tokv = idx_v[pl.ds(g * _L, _L)]
            for j in range(_L):
                tok = tokv[j]
                row = g * 8 + j // 2
                for k in range(D // _L):
                    widx_v[row, pl.ds((j % 2) * D + k * _L, _L)] = (
                        dcol + (k * _L * V + tok)
                    )
            return 0

        lax.fori_loop(0, b_per_w // _L, build, 0)

        def fire(r):
            pltpu.async_copy(
                table_hbm.at[widx_v.at[r]],
                land_v.at[pl.ds(r * 128, 128)],
                gsem,
            )

        def drain1():
            pltpu.make_async_copy(
                table_hbm.at[widx_v.at[0]], land_v.at[pl.ds(0, 128)], gsem
            ).wait()

        def gbody(r, _):
            fire(r)

            @pl.when(r >= _LAG)
            def _():
                drain1()

            return 0

        lax.fori_loop(0, n_rows, gbody, 0)

        def tail(r, _):
            drain1()
            return 0

        lax.fori_loop(0, _LAG, tail, 0)

        pltpu.async_copy(
            land_v, out_hbm.at[pl.ds(base * D, n_words)], osem
        ).wait()

    return gather_kernel


def kernel(token_ids, E):
    B = token_ids.shape[0]
    V, D = E.shape
    idx = token_ids.astype(jnp.int32)
    flat = E.T.reshape(-1)
    out_flat = _build(B, V, D)(idx, flat)
    return out_flat.reshape(B, D)


# speedup vs baseline: 18.7879x; 18.7879x over previous
import functools

import jax
import jax.numpy as jnp
from jax import lax
from jax.experimental import pallas as pl
from jax.experimental.pallas import tpu as pltpu
from jax.experimental.pallas import tpu_sc as plsc

_NUM_CORES = 2
_NUM_SUBCORES = 16
_NUM_WORKERS = _NUM_CORES * _NUM_SUBCORES
_L = 16
_CHUNK = 32


@functools.lru_cache(maxsize=None)
def _build(B, V, D):
    b_per_w = B // _NUM_WORKERS
    n_iters = b_per_w // (2 * _CHUNK)
    mesh = plsc.VectorSubcoreMesh(core_axis_name="c", subcore_axis_name="s")

    @functools.partial(
        pl.kernel,
        mesh=mesh,
        out_type=jax.ShapeDtypeStruct((B, D), jnp.float32),
        scratch_types=[
            pltpu.VMEM((b_per_w,), jnp.int32),
            pltpu.VMEM((2, _CHUNK, 8, D), jnp.float32),
            pltpu.VMEM((2 * _CHUNK, D), jnp.float32),
            pltpu.SemaphoreType.DMA,
            pltpu.SemaphoreType.DMA,
            pltpu.SemaphoreType.DMA,
            pltpu.SemaphoreType.DMA,
        ],
    )
    def gather_kernel(idx_hbm, table_hbm, out_hbm, idx_v, land_v, row_v,
                      isem, gsem0, gsem1, osem):
        gsems = (gsem0, gsem1)
        wid = lax.axis_index("s") * _NUM_CORES + lax.axis_index("c")
        base = wid * b_per_w
        pltpu.async_copy(idx_hbm.at[pl.ds(base, b_per_w)], idx_v, isem).wait()

        def fire(c, buf):
            for g in range(_CHUNK // _L):
                grpv = jax.lax.shift_right_logical(
                    idx_v[pl.ds(c * _CHUNK + g * _L, _L)], 3
                )
                for j in range(_L):
                    pltpu.async_copy(
                        table_hbm.at[grpv[j]],
                        land_v.at[buf, g * _L + j],
                        gsems[buf],
                    )

        def drain(buf):
            for _ in range(_CHUNK):
                pltpu.make_async_copy(
                    table_hbm.at[0], land_v.at[0, 0], gsems[buf]
                ).wait()

        def owait():
            pltpu.make_async_copy(
                row_v, out_hbm.at[pl.ds(0, 2 * _CHUNK)], osem
            ).wait()

        def select(c, buf, half):
            for g in range(_CHUNK // _L):
                tokv = idx_v[pl.ds(c * _CHUNK + g * _L, _L)]
                rv = tokv & 7
                for j in range(_L):
                    r = rv[j]
                    t = half * _CHUNK + g * _L + j
                    for k in range(D // _L):
                        row_v[t, pl.ds(k * _L, _L)] = land_v[
                            buf, g * _L + j, r, pl.ds(k * _L, _L)
                        ]

        fire(0, 0)
        fire(1, 1)

        def body(i, _):
            @pl.when(i >= 1)
            def _():
                owait()

            drain(0)
            select(2 * i, 0, 0)

            @pl.when(i < n_iters - 1)
            def _():
                fire(2 * i + 2, 0)

            drain(1)
            select(2 * i + 1, 1, 1)

            @pl.when(i < n_iters - 1)
            def _():
                fire(2 * i + 3, 1)

            dst = pl.multiple_of(base + i * 2 * _CHUNK, 8)
            pltpu.async_copy(
                row_v, out_hbm.at[pl.ds(dst, 2 * _CHUNK)], osem
            )
            return 0

        lax.fori_loop(0, n_iters, body, 0)
        owait()

    return gather_kernel


def kernel(token_ids, E):
    B = token_ids.shape[0]
    V, D = E.shape
    idx = token_ids.astype(jnp.int32)
    E3 = E.reshape(V // 8, 8, D)
    return _build(B, V, D)(idx, E3)
